# baseline (speedup 1.0000x reference)
import functools
import math

import jax
import jax.numpy as jnp
from jax import lax
from jax.experimental import pallas as pl
from jax.experimental.pallas import tpu as pltpu
from jax.experimental.pallas import tpu_sc as plsc

N = 10000
E = 160000
D = 128

NC = 2
NS = 16
L = 16
NW = NC * NS
G = D // L
CHUNK = 64
RCH = E // CHUNK
CPT = 80
PCH = NW * CPT
NROW = N + 8
RPT = 624
ZREM = NROW - NS * RPT
OREM = N - NS * RPT
LOG2PI = math.log(2.0 * math.pi)


def _sc_layer_body(src_hbm, dst_hbm, table_hbm, eps_hbm, mu_hbm, sig_hbm,
                   part_hbm, nll_hbm,
                   srcs_v, dsts_v, rows0, rows1, eps0, eps1,
                   mu_v, sig_v, nll_v, hacc_sh,
                   g0, g1, e0, e1, s0, s1):
    rows = (rows0, rows1)
    epsb = (eps0, eps1)
    gsem = (g0, g1)
    esem = (e0, e1)
    ssem = (s0, s1)
    cc = lax.axis_index("c")
    ss = lax.axis_index("s")
    wid = cc * NS + ss
    crow0 = wid * CPT

    pltpu.sync_copy(mu_hbm, mu_v)
    pltpu.sync_copy(sig_hbm, sig_v)
    mu_r = [mu_v[g, :] for g in range(G)]
    sig_r = [sig_v[g, :] for g in range(G)]

    def zero_row(r, carry):
        for g in range(G):
            rows0[r, pl.ds(g * L, L)] = jnp.zeros((L,), jnp.float32)
        return carry
    lax.fori_loop(0, CHUNK, zero_row, 0)
    row0 = ss * RPT
    for k in range(RPT // CHUNK):
        pltpu.sync_copy(rows0, hacc_sh.at[pl.ds(row0 + k * CHUNK, CHUNK)])
    zrem = RPT - (RPT // CHUNK) * CHUNK
    if zrem:
        pltpu.sync_copy(rows0.at[pl.ds(0, zrem)],
                        hacc_sh.at[pl.ds(row0 + (RPT // CHUNK) * CHUNK, zrem)])

    @pl.when(ss == NS - 1)
    def _zero_leftover():
        pltpu.sync_copy(rows0.at[pl.ds(0, ZREM)],
                        hacc_sh.at[pl.ds(NS * RPT, ZREM)])

    pltpu.sync_copy(src_hbm.at[pl.ds(crow0 * CHUNK, CPT * CHUNK)], srcs_v)
    pltpu.sync_copy(dst_hbm.at[pl.ds(crow0, CPT)], dsts_v)
    plsc.subcore_barrier()

    def issue(c, b):
        gc = crow0 + c
        ec = jnp.minimum(gc, RCH - 1) * CHUNK
        pltpu.async_copy(table_hbm.at[srcs_v.at[pl.ds(c * CHUNK, CHUNK)]],
                         rows[b], gsem[b])
        pltpu.async_copy(eps_hbm.at[pl.ds(ec, CHUNK)], epsb[b], esem[b])

    def wait_in(c, b):
        pltpu.make_async_copy(table_hbm.at[srcs_v.at[pl.ds(c * CHUNK, CHUNK)]],
                              rows[b], gsem[b]).wait()
        pltpu.make_async_copy(eps_hbm.at[pl.ds(0, CHUNK)], epsb[b],
                              esem[b]).wait()

    def wait_scat(c, b):
        pltpu.make_async_copy(rows[b], hacc_sh.at[dsts_v.at[c]],
                              ssem[b]).wait()

    def scale_chunk(rows_ref, eps_ref):
        def row_body(r, acc):
            for g in range(G):
                sl = pl.ds(g * L, L)
                a = mu_r[g] + sig_r[g] * eps_ref[r, sl]
                t = a - 1.0
                acc = acc + t * t
                rows_ref[r, sl] = rows_ref[r, sl] * a
            return acc
        return lax.fori_loop(0, CHUNK, row_body, jnp.zeros((L,), jnp.float32))

    def step(c, b, acc):
        wait_in(c, b)

        @pl.when(c >= 1)
        def _drain_prev():
            wait_scat(c - 1, 1 - b)

        @pl.when(c + 1 < CPT)
        def _prefetch():
            issue(c + 1, 1 - b)

        gc = crow0 + c
        mask = jnp.where(gc < RCH, 1.0, 0.0).astype(jnp.float32)
        acc = acc + mask * scale_chunk(rows[b], epsb[b])
        pltpu.async_copy(rows[b], hacc_sh.at[dsts_v.at[c]], ssem[b], add=True)
        return acc

    issue(jnp.int32(0), 0)

    def pair_body(t, acc):
        for k in range(2):
            acc = step(t * 2 + k, k, acc)
        return acc

    acc = lax.fori_loop(0, CPT // 2, pair_body,
                        jnp.zeros((L,), jnp.float32))
    wait_scat(jnp.int32(CPT - 1), (CPT - 1) % 2)

    nll_v[...] = acc
    pltpu.sync_copy(nll_v, nll_hbm.at[pl.ds(wid * L, L)])

    plsc.subcore_barrier()
    pltpu.sync_copy(hacc_sh.at[pl.ds(row0, RPT)],
                    part_hbm.at[pl.ds(cc * N + row0, RPT)])

    @pl.when(ss == NS - 1)
    def _read_leftover():
        pltpu.sync_copy(hacc_sh.at[pl.ds(NS * RPT, OREM)],
                        part_hbm.at[pl.ds(cc * N + NS * RPT, OREM)])


def _make_sc_layer():
    mesh = plsc.VectorSubcoreMesh(core_axis_name="c", subcore_axis_name="s",
                                  num_cores=NC, num_subcores=NS)
    return pl.kernel(
        _sc_layer_body,
        out_type=(
            jax.ShapeDtypeStruct((NC * N, D), jnp.float32),
            jax.ShapeDtypeStruct((NW * L,), jnp.float32),
        ),
        mesh=mesh,
        scratch_types=[
            pltpu.VMEM((CPT * CHUNK,), jnp.int32),
            pltpu.VMEM((CPT, CHUNK), jnp.int32),
            pltpu.VMEM((CHUNK, D), jnp.float32),
            pltpu.VMEM((CHUNK, D), jnp.float32),
            pltpu.VMEM((CHUNK, D), jnp.float32),
            pltpu.VMEM((CHUNK, D), jnp.float32),
            pltpu.VMEM((G, L), jnp.float32),
            pltpu.VMEM((G, L), jnp.float32),
            pltpu.VMEM((L,), jnp.float32),
            pltpu.VMEM_SHARED((NROW, D), jnp.float32),
            pltpu.SemaphoreType.DMA,
            pltpu.SemaphoreType.DMA,
            pltpu.SemaphoreType.DMA,
            pltpu.SemaphoreType.DMA,
            pltpu.SemaphoreType.DMA,
            pltpu.SemaphoreType.DMA,
        ],
    )


_ROWS_BLK = 1000
_GRID = N // _ROWS_BLK


def _tc_mid_body(p0_ref, p1_ref, w_ref, b_ref, o_ref):
    s = p0_ref[...] + p1_ref[...]
    y = jnp.dot(s, w_ref[...], preferred_element_type=jnp.float32)
    o_ref[...] = jnp.maximum(y + b_ref[...], 0.0)


def _tc_final_body(p0_ref, p1_ref, w_ref, b_ref, n1_ref, n2_ref,
                   o_ref, nll_ref):
    s = p0_ref[...] + p1_ref[...]
    y = jnp.dot(s, w_ref[...], preferred_element_type=jnp.float32)
    o_ref[...] = y + b_ref[...]

    @pl.when(pl.program_id(0) == 0)
    def _():
        tot = jnp.sum(n1_ref[...]) + jnp.sum(n2_ref[...])
        nll_ref[...] = jnp.reshape(0.5 * tot / float(E * D) + LOG2PI, (1, 1))


def _tc_mid(parts, w, b):
    return pl.pallas_call(
        _tc_mid_body,
        grid=(_GRID,),
        in_specs=[
            pl.BlockSpec((_ROWS_BLK, D), lambda i: (i, 0)),
            pl.BlockSpec((_ROWS_BLK, D), lambda i: (i + _GRID, 0)),
            pl.BlockSpec((D, D), lambda i: (0, 0)),
            pl.BlockSpec((1, D), lambda i: (0, 0)),
        ],
        out_specs=pl.BlockSpec((_ROWS_BLK, D), lambda i: (i, 0)),
        out_shape=jax.ShapeDtypeStruct((N, D), jnp.float32),
    )(parts, parts, w, b)


def _tc_final(parts, w, b, n1, n2):
    return pl.pallas_call(
        _tc_final_body,
        grid=(_GRID,),
        in_specs=[
            pl.BlockSpec((_ROWS_BLK, D), lambda i: (i, 0)),
            pl.BlockSpec((_ROWS_BLK, D), lambda i: (i + _GRID, 0)),
            pl.BlockSpec((D, D), lambda i: (0, 0)),
            pl.BlockSpec((1, D), lambda i: (0, 0)),
            pl.BlockSpec((NW * L // D, D), lambda i: (0, 0)),
            pl.BlockSpec((NW * L // D, D), lambda i: (0, 0)),
        ],
        out_specs=[
            pl.BlockSpec((_ROWS_BLK, D), lambda i: (i, 0)),
            pl.BlockSpec((1, 1), lambda i: (0, 0)),
        ],
        out_shape=[
            jax.ShapeDtypeStruct((N, D), jnp.float32),
            jax.ShapeDtypeStruct((1, 1), jnp.float32),
        ],
    )(parts, parts, w, b, n1, n2)


def kernel(x, edge_index, W0, b0, W1, b1, a_mu, a_log_sigma,
           a_mu_first, a_log_sigma_first, eps_first, eps_rest):
    src1d = jnp.pad(edge_index[0], (0, (PCH - RCH) * CHUNK))
    dst2d = jnp.pad(edge_index[1].reshape(RCH, CHUNK),
                    ((0, PCH - RCH), (0, 0)), constant_values=N)
    sc_layer = _make_sc_layer()

    mu0 = a_mu_first.reshape(G, L)
    sig0 = a_log_sigma_first.reshape(G, L)
    part0, nllp0 = sc_layer(src1d, dst2d, x, eps_first, mu0, sig0)
    h = _tc_mid(part0, W0, b0.reshape(1, D))

    mu1 = a_mu[0].reshape(G, L)
    sig1 = a_log_sigma[0].reshape(G, L)
    eps1 = eps_rest.reshape(E, D)
    part1, nllp1 = sc_layer(src1d, dst2d, h, eps1, mu1, sig1)
    out, nll = _tc_final(part1, W1, b1.reshape(1, D),
                         nllp0.reshape(NW * L // D, D),
                         nllp1.reshape(NW * L // D, D))
    return (out, nll[0, 0])
